# Initial kernel scaffold; baseline (speedup 1.0000x reference)
import jax
import jax.numpy as jnp
from jax.experimental import pallas as pl


def kernel(x, edge_index, W_gcn, b_gcn, W_ih, W_hh, b_ih, b_hh, h0):
    raise NotImplementedError("write your pallas kernel here")


# speedup vs baseline: 4.8124x; 4.8124x over previous
import functools

import jax
import jax.numpy as jnp
from jax.experimental import pallas as pl
from jax.experimental.pallas import tpu as pltpu

N = 10000
E = 160000
D = 256
H = 256
H3 = 768
RB = 1000
NBLK = N // RB
PLANE_ROWS = 5120
HALF = 5000

_PREC = jax.lax.Precision.HIGHEST


def _scale_kernel(deg_ref, x_ref, w_ref, y_ref, dinv_ref):
    deg = deg_ref[0, :, 0:1] + deg_ref[1, :, 0:1] + 1.0
    dinv = jax.lax.rsqrt(deg)
    xw = jax.lax.dot_general(x_ref[...], w_ref[...], (((1,), (1,)), ((), ())),
                             preferred_element_type=jnp.float32,
                             precision=_PREC)
    y_ref[...] = dinv * xw
    dinv_ref[...] = dinv


def _gi_kernel(z_ref, y_ref, dinv_ref, bg_ref, wt_ref, bias_ref, gi_ref):
    g = dinv_ref[...] * (z_ref[0] + y_ref[...]) + bg_ref[...]
    gi_ref[...] = jax.lax.dot_general(g, wt_ref[...], (((1,), (0,)), ((), ())),
                                      preferred_element_type=jnp.float32,
                                      precision=_PREC) + bias_ref[...]


def _gru_kernel(h0_ref, gi_ref, wt_ref, bhn_ref, out_ref, h_scr):
    @pl.when(pl.program_id(0) == 0)
    def _():
        h_scr[...] = h0_ref[...]

    def body(t, h):
        gih = gi_ref[pl.ds(t, 1), :]
        gh = jax.lax.dot_general(h, wt_ref[...], (((1,), (0,)), ((), ())),
                                 preferred_element_type=jnp.float32,
                                 precision=_PREC)
        rz = jax.nn.sigmoid(gih[:, 0:2 * H] + gh[:, 0:2 * H])
        r = rz[:, 0:H]
        zg = rz[:, H:2 * H]
        n = jnp.tanh(gih[:, 2 * H:] + r * (gh[:, 2 * H:] + bhn_ref[...]))
        hn = zg * (h - n) + n
        out_ref[pl.ds(t, 1), :] = hn
        return hn

    h_scr[...] = jax.lax.fori_loop(0, RB, body, h_scr[...])


def kernel(x, edge_index, W_gcn, b_gcn, W_ih, W_hh, b_ih, b_hh, h0):
    src = edge_index[0]
    dst = edge_index[1]

    cnt = jnp.zeros((N,), jnp.float32).at[dst].add(1.0)
    deg_planes = jnp.zeros((2, 10016, 16), jnp.float32).at[0, :N, 0].set(cnt)

    y, dinv = pl.pallas_call(
        _scale_kernel,
        grid=(NBLK,),
        in_specs=[
            pl.BlockSpec((2, RB, 16), lambda i: (0, i, 0)),
            pl.BlockSpec((RB, D), lambda i: (i, 0)),
            pl.BlockSpec((D, D), lambda i: (0, 0)),
        ],
        out_specs=[
            pl.BlockSpec((RB, D), lambda i: (i, 0)),
            pl.BlockSpec((RB, 1), lambda i: (i, 0)),
        ],
        out_shape=[
            jax.ShapeDtypeStruct((N, D), jnp.float32),
            jax.ShapeDtypeStruct((N, 1), jnp.float32),
        ],
    )(deg_planes, x, W_gcn)

    zc = jnp.zeros((N, D), jnp.float32).at[dst].add(y[src])
    z_planes = (jnp.zeros((2, PLANE_ROWS, D), jnp.float32)
                .at[0, :HALF].set(zc[:HALF])
                .at[1, :HALF].set(zc[HALF:]))

    bias1 = (b_ih + jnp.concatenate([b_hh[:2 * H], jnp.zeros((H,), jnp.float32)]))[None]
    gi = pl.pallas_call(
        _gi_kernel,
        grid=(NBLK,),
        in_specs=[
            pl.BlockSpec((1, RB, D), lambda i: (i // 5, i % 5, 0)),
            pl.BlockSpec((RB, D), lambda i: (i, 0)),
            pl.BlockSpec((RB, 1), lambda i: (i, 0)),
            pl.BlockSpec((1, D), lambda i: (0, 0)),
            pl.BlockSpec((D, H3), lambda i: (0, 0)),
            pl.BlockSpec((1, H3), lambda i: (0, 0)),
        ],
        out_specs=pl.BlockSpec((RB, H3), lambda i: (i, 0)),
        out_shape=jax.ShapeDtypeStruct((N, H3), jnp.float32),
    )(z_planes, y, dinv, b_gcn[None], W_ih.T, bias1)

    seq = pl.pallas_call(
        _gru_kernel,
        grid=(NBLK,),
        in_specs=[
            pl.BlockSpec((1, H), lambda i: (0, 0)),
            pl.BlockSpec((RB, H3), lambda i: (i, 0)),
            pl.BlockSpec((H, H3), lambda i: (0, 0)),
            pl.BlockSpec((1, H), lambda i: (0, 0)),
        ],
        out_specs=pl.BlockSpec((RB, H), lambda i: (i, 0)),
        out_shape=jax.ShapeDtypeStruct((N, H), jnp.float32),
        scratch_shapes=[pltpu.VMEM((1, H), jnp.float32)],
    )(h0[0], gi, W_hh.T, b_hh[2 * H:][None])

    out = seq[None]
    h_n = seq[N - 1:N][None]
    return out, h_n


# speedup vs baseline: 8.8929x; 1.8479x over previous
import functools

import jax
import jax.numpy as jnp
from jax.experimental import pallas as pl
from jax.experimental.pallas import tpu as pltpu

N = 10000
E = 160000
D = 256
H = 256
H3 = 768
RB = 1000
NBLK = N // RB
PLANE_ROWS = 5120
HALF = 5000

_PREC = jax.lax.Precision.HIGHEST


def _scale_kernel(deg_ref, x_ref, w_ref, y_ref, dinv_ref):
    deg = deg_ref[0, :, 0:1] + deg_ref[1, :, 0:1] + 1.0
    dinv = jax.lax.rsqrt(deg)
    xw = jax.lax.dot_general(x_ref[...], w_ref[...], (((1,), (1,)), ((), ())),
                             preferred_element_type=jnp.float32,
                             precision=_PREC)
    y_ref[...] = dinv * xw
    dinv_ref[...] = dinv


def _gi_kernel(z_ref, y_ref, dinv_ref, bg_ref, wt_ref, bias_ref, gi_ref):
    g = dinv_ref[...] * (z_ref[0] + y_ref[...]) + bg_ref[...]
    gi_ref[...] = jax.lax.dot_general(g, wt_ref[...], (((1,), (0,)), ((), ())),
                                      preferred_element_type=jnp.float32,
                                      precision=_PREC) + bias_ref[...]


def _gru_kernel(h0_ref, gi_ref, wt_ref, bhn_ref, out_ref, h_scr):
    @pl.when(pl.program_id(0) == 0)
    def _():
        h_scr[...] = h0_ref[...]

    def body(t, h):
        gih = gi_ref[pl.ds(t, 1), :]
        gh = jax.lax.dot_general(h.astype(jnp.bfloat16), wt_ref[...],
                                 (((1,), (0,)), ((), ())),
                                 preferred_element_type=jnp.float32)
        rz = jax.nn.sigmoid(gih[:, 0:2 * H] + gh[:, 0:2 * H])
        r = rz[:, 0:H]
        zg = rz[:, H:2 * H]
        n = jnp.tanh(gih[:, 2 * H:] + r * (gh[:, 2 * H:] + bhn_ref[...]))
        hn = zg * (h - n) + n
        out_ref[pl.ds(t, 1), :] = hn
        return hn

    h_scr[...] = jax.lax.fori_loop(0, RB, body, h_scr[...])


def kernel(x, edge_index, W_gcn, b_gcn, W_ih, W_hh, b_ih, b_hh, h0):
    src = edge_index[0]
    dst = edge_index[1]

    cnt = jnp.zeros((N,), jnp.float32).at[dst].add(1.0)
    deg_planes = jnp.zeros((2, 10016, 16), jnp.float32).at[0, :N, 0].set(cnt)

    y, dinv = pl.pallas_call(
        _scale_kernel,
        grid=(NBLK,),
        in_specs=[
            pl.BlockSpec((2, RB, 16), lambda i: (0, i, 0)),
            pl.BlockSpec((RB, D), lambda i: (i, 0)),
            pl.BlockSpec((D, D), lambda i: (0, 0)),
        ],
        out_specs=[
            pl.BlockSpec((RB, D), lambda i: (i, 0)),
            pl.BlockSpec((RB, 1), lambda i: (i, 0)),
        ],
        out_shape=[
            jax.ShapeDtypeStruct((N, D), jnp.float32),
            jax.ShapeDtypeStruct((N, 1), jnp.float32),
        ],
    )(deg_planes, x, W_gcn)

    zc = jnp.zeros((N, D), jnp.float32).at[dst].add(y[src])
    z_planes = (jnp.zeros((2, PLANE_ROWS, D), jnp.float32)
                .at[0, :HALF].set(zc[:HALF])
                .at[1, :HALF].set(zc[HALF:]))

    bias1 = (b_ih + jnp.concatenate([b_hh[:2 * H], jnp.zeros((H,), jnp.float32)]))[None]
    gi = pl.pallas_call(
        _gi_kernel,
        grid=(NBLK,),
        in_specs=[
            pl.BlockSpec((1, RB, D), lambda i: (i // 5, i % 5, 0)),
            pl.BlockSpec((RB, D), lambda i: (i, 0)),
            pl.BlockSpec((RB, 1), lambda i: (i, 0)),
            pl.BlockSpec((1, D), lambda i: (0, 0)),
            pl.BlockSpec((D, H3), lambda i: (0, 0)),
            pl.BlockSpec((1, H3), lambda i: (0, 0)),
        ],
        out_specs=pl.BlockSpec((RB, H3), lambda i: (i, 0)),
        out_shape=jax.ShapeDtypeStruct((N, H3), jnp.float32),
    )(z_planes, y, dinv, b_gcn[None], W_ih.T, bias1)

    seq = pl.pallas_call(
        _gru_kernel,
        grid=(NBLK,),
        in_specs=[
            pl.BlockSpec((1, H), lambda i: (0, 0)),
            pl.BlockSpec((RB, H3), lambda i: (i, 0)),
            pl.BlockSpec((H, H3), lambda i: (0, 0)),
            pl.BlockSpec((1, H), lambda i: (0, 0)),
        ],
        out_specs=pl.BlockSpec((RB, H), lambda i: (i, 0)),
        out_shape=jax.ShapeDtypeStruct((N, H), jnp.float32),
        scratch_shapes=[pltpu.VMEM((1, H), jnp.float32)],
    )(h0[0], gi, W_hh.T.astype(jnp.bfloat16), b_hh[2 * H:][None])

    out = seq[None]
    h_n = seq[N - 1:N][None]
    return out, h_n
